# baseline (speedup 1.0000x reference)
import functools

import jax
import jax.numpy as jnp
from jax import lax
from jax.experimental import pallas as pl
from jax.experimental.pallas import tpu as pltpu
from jax.experimental.pallas import tpu_sc as plsc


@functools.cache
def _detile_idx_kernel(B: int, L: int):
    info = plsc.get_sparse_core_info()
    NC, NS = info.num_cores, info.num_subcores
    NW = NC * NS
    NB = B // 128
    assert NB % NW == 0
    BPW = NB // NW
    C8 = (L + 7) // 8 * 8
    full = L // 8
    rem = L - full * 8
    n_tiles = C8 // 8

    mesh = plsc.VectorSubcoreMesh(core_axis_name="c", subcore_axis_name="s")

    @functools.partial(
        pl.kernel,
        mesh=mesh,
        out_type=jax.ShapeDtypeStruct((NB * C8 * 128,), jnp.int32),
        scratch_types=[
            pltpu.VMEM((C8, 128), jnp.int32),
            pltpu.VMEM((C8 * 128,), jnp.int32),
            pltpu.SemaphoreType.DMA,
            pltpu.SemaphoreType.DMA,
        ],
    )
    def k(idxt_hbm, out_hbm, st_v, f_v, sem_in, sem_out):
        wid = lax.axis_index("s") * NC + lax.axis_index("c")
        for blk in range(BPW):
            t = wid * BPW + blk
            cps = []
            for a in range(n_tiles):
                h = 8 if (a < full) else rem
                cps.append(pltpu.async_copy(
                    idxt_hbm.at[pl.ds(a * 8, h), pl.ds(t * 128, 128)],
                    st_v.at[pl.ds(a * 8, h), :], sem_in))
            for cp in cps:
                cp.wait()
            for j in range(L):
                for kk in range(8):
                    f_v[pl.ds(j * 128 + 16 * kk, 16)] = st_v[j, pl.ds(16 * kk, 16)]
            pltpu.async_copy(
                f_v, out_hbm.at[pl.ds(t * C8 * 128, C8 * 128)], sem_out).wait()

    return k


@functools.cache
def _pooled_sum_kernel(B: int, L: int, V: int, D: int, CB: int):
    info = plsc.get_sparse_core_info()
    NC, NS = info.num_cores, info.num_subcores
    NW = NC * NS
    NB = B // 128
    assert NB % NW == 0
    BPW = NB // NW
    C8 = (L + 7) // 8 * 8
    CPB = 128 // CB
    assert CPB * CB == 128 and CB % 16 == 0
    assert L % 2 == 0

    mesh = plsc.VectorSubcoreMesh(core_axis_name="c", subcore_axis_name="s")

    @functools.partial(
        pl.kernel,
        mesh=mesh,
        out_type=jax.ShapeDtypeStruct((B, D), jnp.float32),
        compiler_params=pltpu.CompilerParams(use_tc_tiling_on_sc=False),
        scratch_types=[
            pltpu.VMEM((C8 * 128,), jnp.int32),
            pltpu.VMEM((CB * L,), jnp.int32),
            pltpu.VMEM((CB * L,), jnp.int32),
            pltpu.VMEM((CB * L, D), jnp.float32),
            pltpu.VMEM((CB * L, D), jnp.float32),
            pltpu.VMEM((CB, D), jnp.float32),
            pltpu.VMEM((CB, D), jnp.float32),
            pltpu.SemaphoreType.DMA,
            pltpu.SemaphoreType.DMA,
        ],
    )
    def k(table_hbm, idx_hbm, out_hbm, st_v, idx0, idx1, rows0, rows1,
          acc0, acc1, sem0, sem1):
        wid = lax.axis_index("s") * NC + lax.axis_index("c")

        def repack_and_fire(cc, idx_v, rows_v, sem):
            col = cc * CB
            for j in range(L):
                for kk in range(CB // 16):
                    idx_v[pl.ds(j * CB + 16 * kk, 16)] = (
                        st_v[pl.ds(j * 128 + col + 16 * kk, 16)])
            pltpu.async_copy(table_hbm.at[idx_v], rows_v, sem)

        def accum_store(t, cc, rows_v, acc_v):
            def batch_body(b, carry2):
                z = jnp.zeros((16,), jnp.float32)
                a0 = a1 = a2 = a3 = z
                for j in range(0, L, 2):
                    a0 = a0 + rows_v[j * CB + b, pl.ds(0, 16)]
                    a1 = a1 + rows_v[j * CB + b, pl.ds(16, 16)]
                    a2 = a2 + rows_v[(j + 1) * CB + b, pl.ds(0, 16)]
                    a3 = a3 + rows_v[(j + 1) * CB + b, pl.ds(16, 16)]
                acc_v[b, pl.ds(0, 16)] = a0 + a2
                acc_v[b, pl.ds(16, 16)] = a1 + a3
                return carry2

            lax.fori_loop(0, CB, batch_body, 0)
            pltpu.sync_copy(acc_v, out_hbm.at[pl.ds(t * 128 + cc * CB, CB)])

        def wait_gather(idx_v, rows_v, sem):
            pltpu.make_async_copy(table_hbm.at[idx_v], rows_v, sem).wait()

        bufs = [(idx0, rows0, acc0, sem0), (idx1, rows1, acc1, sem1)]

        def block_body(blk, carry):
            t = wid * BPW + blk
            pltpu.sync_copy(idx_hbm.at[pl.ds(t * C8 * 128, C8 * 128)], st_v)
            repack_and_fire(0, *bufs[0][:2], bufs[0][3])
            if CPB > 1:
                repack_and_fire(1, *bufs[1][:2], bufs[1][3])
            for cc in range(CPB):
                ib, rb, ab, sb = bufs[cc % 2]
                wait_gather(ib, rb, sb)
                accum_store(t, cc, rb, ab)
                if cc + 2 < CPB:
                    repack_and_fire(cc + 2, ib, rb, sb)
            return carry

        lax.fori_loop(0, BPW, block_body, 0)

    return k


def _head_body(s1_ref, s2_ref, nl_ref, wl_ref, w1_ref, b1_ref, w2_ref,
               b2_ref, o_ref):
    x1 = s1_ref[...] / nl_ref[...]
    x2 = s2_ref[...] / wl_ref[...]
    h = jnp.tanh(jnp.concatenate([x1, x2], axis=1))
    u = lax.dot_general(h, w1_ref[...], (((1,), (1,)), ((), ())),
                        preferred_element_type=jnp.float32) + b1_ref[...]
    o_ref[...] = lax.dot_general(w2_ref[...], u, (((1,), (1,)), ((), ())),
                                 preferred_element_type=jnp.float32) + b2_ref[...]


def _head(s1, s2, ngram_len, word_len, W1, b1, W2, b2):
    B, D = s1.shape
    OUTV, OUTD = W2.shape
    BM = 512
    grid = (B // BM,)
    nl = ngram_len.reshape(B, 1)
    wl = word_len.reshape(B, 1)
    yt = pl.pallas_call(
        _head_body,
        grid=grid,
        in_specs=[
            pl.BlockSpec((BM, D), lambda i: (i, 0)),
            pl.BlockSpec((BM, D), lambda i: (i, 0)),
            pl.BlockSpec((BM, 1), lambda i: (i, 0)),
            pl.BlockSpec((BM, 1), lambda i: (i, 0)),
            pl.BlockSpec((OUTD, 2 * D), lambda i: (0, 0)),
            pl.BlockSpec((1, OUTD), lambda i: (0, 0)),
            pl.BlockSpec((OUTV, OUTD), lambda i: (0, 0)),
            pl.BlockSpec((OUTV, 1), lambda i: (0, 0)),
        ],
        out_specs=pl.BlockSpec((OUTV, BM), lambda i: (0, i)),
        out_shape=jax.ShapeDtypeStruct((OUTV, B), jnp.float32),
    )(s1, s2, nl, wl, W1, b1.reshape(1, OUTD), W2, b2.reshape(OUTV, 1))
    return yt.T


def kernel(words, word_len, ngrams, ngram_len, ngram_table, word_table,
           W1, b1, W2, b2):
    B, LW = words.shape
    _, LN = ngrams.shape
    WV, WD = word_table.shape
    NV, ND = ngram_table.shape

    ngrams_t = ngrams.astype(jnp.int32).T
    words_t = words.astype(jnp.int32).T

    ngflat = _detile_idx_kernel(B, LN)(ngrams_t)
    wdflat = _detile_idx_kernel(B, LW)(words_t)
    s1 = _pooled_sum_kernel(B, LN, NV, ND, 32)(ngram_table, ngflat)
    s2 = _pooled_sum_kernel(B, LW, WV, WD, 64)(word_table, wdflat)
    return _head(s1, s2, ngram_len, word_len, W1, b1, W2, b2)
